# Initial kernel scaffold; baseline (speedup 1.0000x reference)
import jax
import jax.numpy as jnp
from jax.experimental import pallas as pl


def kernel(x, edge_index, edge_type, w_l0, b_l0, w_00, b_00, w_10, b_10, w_l1, b_l1, w_01, b_01, w_11, b_11):
    raise NotImplementedError("write your pallas kernel here")


# speedup vs baseline: 20.7751x; 20.7751x over previous
import functools

import jax
import jax.numpy as jnp
from jax import lax
from jax.experimental import pallas as pl
from jax.experimental.pallas import tpu as pltpu
from jax.experimental.pallas import tpu_sc as plsc

N = 10000
E = 640000
IN_CH = 128
HID = 64

NC = 2
NS = 16
NW = NC * NS
CHUNK = 128
SUPER = 16
NSUPER = 10
EPAD = NW * NSUPER * SUPER * CHUNK
ROWS2D = EPAD // CHUNK
ROWS_PER_W = NSUPER * SUPER

ACC_ROWS = 10496
TRASH0 = 10240
ZCH = ACC_ROWS // NS // 2
NOUT = 10240
OCH = NOUT // NS // 5

BR = 1000


def _make_sc_scatter(rel):
    mesh = plsc.VectorSubcoreMesh(
        core_axis_name="c", subcore_axis_name="s", num_cores=NC, num_subcores=NS
    )

    @functools.partial(
        pl.kernel,
        out_type=jax.ShapeDtypeStruct((NC * NOUT, HID), jnp.float32),
        mesh=mesh,
        compiler_params=pltpu.CompilerParams(use_tc_tiling_on_sc=False),
        scratch_types=[
            pltpu.VMEM((SUPER, CHUNK), jnp.int32),
            pltpu.VMEM((SUPER, CHUNK), jnp.int32),
            pltpu.VMEM((SUPER, CHUNK), jnp.int32),
            pltpu.VMEM((SUPER, CHUNK), jnp.int32),
            pltpu.VMEM((CHUNK, HID), jnp.float32),
            pltpu.VMEM((CHUNK, HID), jnp.float32),
            pltpu.VMEM((ZCH, HID), jnp.float32),
            pltpu.VMEM((OCH, HID), jnp.float32),
            pltpu.VMEM_SHARED((ACC_ROWS, HID), jnp.float32),
            pltpu.SemaphoreType.DMA,
            pltpu.SemaphoreType.DMA,
        ],
    )
    def sc_scatter(hl, tgt2, src2, typ2, zrows, out,
                   tgtbuf, srcbuf, typbuf, dstbuf, rows0, rows1, zbuf, obuf,
                   acc, sem0, sem1):
        cid = lax.axis_index("c")
        sid = lax.axis_index("s")
        wid = sid * NC + cid

        pltpu.sync_copy(zrows, zbuf)
        for t in range(2):
            pltpu.sync_copy(zbuf, acc.at[pl.ds(sid * (2 * ZCH) + t * ZCH, ZCH)])
        plsc.subcore_barrier()

        lanes = lax.iota(jnp.int32, 16)
        trash = TRASH0 + sid * 16 + lanes
        rows = (rows0, rows1)
        sems = (sem0, sem1)

        def body(s, carry):
            row0 = wid * ROWS_PER_W + s * SUPER
            pltpu.sync_copy(tgt2.at[pl.ds(row0, SUPER)], tgtbuf)
            pltpu.sync_copy(src2.at[pl.ds(row0, SUPER)], srcbuf)
            pltpu.sync_copy(typ2.at[pl.ds(row0, SUPER)], typbuf)
            for r in range(SUPER):
                for k in range(CHUNK // 16):
                    sl = pl.ds(k * 16, 16)
                    dstbuf[r, sl] = jnp.where(
                        typbuf[r, sl] == rel, srcbuf[r, sl], trash
                    )
            descs = [None, None]
            descs[0] = pltpu.async_copy(hl.at[tgtbuf.at[0]], rows0, sem0)
            for j in range(SUPER):
                b = j % 2
                if j + 1 < SUPER:
                    nb = (j + 1) % 2
                    descs[nb] = pltpu.async_copy(
                        hl.at[tgtbuf.at[j + 1]], rows[nb], sems[nb]
                    )
                descs[b].wait()
                pltpu.sync_copy(rows[b], acc.at[dstbuf.at[j]], add=True)
            return carry

        lax.fori_loop(0, NSUPER, body, 0)

        plsc.subcore_barrier()
        for t in range(5):
            r0 = sid * (5 * OCH) + t * OCH
            pltpu.sync_copy(acc.at[pl.ds(r0, OCH)], obuf)
            pltpu.sync_copy(obuf, out.at[pl.ds(cid * NOUT + r0, OCH)])

    return sc_scatter


_sc_scatter_rel0 = _make_sc_scatter(0)
_sc_scatter_rel1 = _make_sc_scatter(1)


def _tc_layer0(x, wcat, bias):
    def body(x_ref, w_ref, b_ref, hl_ref, d_ref):
        z = jnp.dot(x_ref[...], w_ref[...], preferred_element_type=jnp.float32)
        hl_ref[...] = z[:, :HID]
        d_ref[...] = z[:, HID:] + b_ref[...]

    return pl.pallas_call(
        body,
        grid=(N // BR,),
        in_specs=[
            pl.BlockSpec((BR, IN_CH), lambda i: (i, 0)),
            pl.BlockSpec((IN_CH, 2 * HID), lambda i: (0, 0)),
            pl.BlockSpec((1, HID), lambda i: (0, 0)),
        ],
        out_specs=[
            pl.BlockSpec((BR, HID), lambda i: (i, 0)),
            pl.BlockSpec((BR, HID), lambda i: (i, 0)),
        ],
        out_shape=[
            jax.ShapeDtypeStruct((N, HID), jnp.float32),
            jax.ShapeDtypeStruct((N, HID), jnp.float32),
        ],
    )(x, wcat, bias)


def _tc_mid(parts, dense, wcat, bias):
    def body(p_ref, d_ref, w_ref, b_ref, hl_ref, d1_ref):
        h1 = jnp.maximum(p_ref[0] + p_ref[1] + d_ref[...], 0.0)
        z = jnp.dot(h1, w_ref[...], preferred_element_type=jnp.float32)
        hl_ref[...] = z[:, :HID]
        d1_ref[...] = z[:, HID:] + b_ref[...]

    return pl.pallas_call(
        body,
        grid=(N // BR,),
        in_specs=[
            pl.BlockSpec((NC, BR, HID), lambda i: (0, i, 0)),
            pl.BlockSpec((BR, HID), lambda i: (i, 0)),
            pl.BlockSpec((HID, 2 * HID), lambda i: (0, 0)),
            pl.BlockSpec((1, HID), lambda i: (0, 0)),
        ],
        out_specs=[
            pl.BlockSpec((BR, HID), lambda i: (i, 0)),
            pl.BlockSpec((BR, HID), lambda i: (i, 0)),
        ],
        out_shape=[
            jax.ShapeDtypeStruct((N, HID), jnp.float32),
            jax.ShapeDtypeStruct((N, HID), jnp.float32),
        ],
    )(parts, dense, wcat, bias)


def _tc_final(parts, dense):
    def body(p_ref, d_ref, o_ref):
        o_ref[...] = jnp.maximum(p_ref[0] + p_ref[1] + d_ref[...], 0.0)

    return pl.pallas_call(
        body,
        grid=(N // BR,),
        in_specs=[
            pl.BlockSpec((NC, BR, HID), lambda i: (0, i, 0)),
            pl.BlockSpec((BR, HID), lambda i: (i, 0)),
        ],
        out_specs=pl.BlockSpec((BR, HID), lambda i: (i, 0)),
        out_shape=jax.ShapeDtypeStruct((N, HID), jnp.float32),
    )(parts, dense)


def kernel(x, edge_index, edge_type, w_l0, b_l0, w_00, b_00, w_10, b_10,
           w_l1, b_l1, w_01, b_01, w_11, b_11):
    src = edge_index[0]
    tgt = edge_index[1]

    pad = EPAD - E
    tgt_p = jnp.concatenate([tgt, jnp.arange(pad, dtype=jnp.int32) % N])
    src_p = jnp.concatenate([src, jnp.zeros((pad,), jnp.int32)])
    typ_p = jnp.concatenate([edge_type, jnp.full((pad,), 2, jnp.int32)])
    tgt2 = tgt_p.reshape(ROWS2D, CHUNK)
    src2 = src_p.reshape(ROWS2D, CHUNK)
    typ2 = typ_p.reshape(ROWS2D, CHUNK)
    zrows = jnp.zeros((ZCH, HID), jnp.float32)

    wcat0 = jnp.concatenate([w_l0, w_00 + w_10], axis=1)
    bias0 = (b_l0 + b_00 + b_10)[None, :]
    wcat1 = jnp.concatenate([w_l1, w_01 + w_11], axis=1)
    bias1 = (b_l1 + b_01 + b_11)[None, :]

    hl0, dense0 = _tc_layer0(x, wcat0, bias0)
    parts0 = _sc_scatter_rel0(hl0, tgt2, src2, typ2, zrows)
    parts0 = parts0.reshape(NC, NOUT, HID)[:, :N]
    hl1, dense1 = _tc_mid(parts0, dense0, wcat1, bias1)
    parts1 = _sc_scatter_rel1(hl1, tgt2, src2, typ2, zrows)
    parts1 = parts1.reshape(NC, NOUT, HID)[:, :N]
    return _tc_final(parts1, dense1)


# speedup vs baseline: 27.9145x; 1.3437x over previous
import functools

import jax
import jax.numpy as jnp
from jax import lax
from jax.experimental import pallas as pl
from jax.experimental.pallas import tpu as pltpu
from jax.experimental.pallas import tpu_sc as plsc

N = 10000
E = 640000
IN_CH = 128
HID = 64

NC = 2
NS = 16
NW = NC * NS
CHUNK = 128
SUPER = 16
NSUPER = 10
EPAD = NW * NSUPER * SUPER * CHUNK
ROWS2D = EPAD // CHUNK
ROWS_PER_W = NSUPER * SUPER

ACC_ROWS = 10144
TRASH0 = 10016
OCH = 208

EDGES_PER_W = ROWS_PER_W * CHUNK
CBUF = EDGES_PER_W + 2 * CHUNK

BR = 1000


def _make_sc_scatter():
    rel = 0
    mesh = plsc.VectorSubcoreMesh(
        core_axis_name="c", subcore_axis_name="s", num_cores=NC, num_subcores=NS
    )

    @functools.partial(
        pl.kernel,
        out_type=jax.ShapeDtypeStruct((NC * N, HID), jnp.float32),
        mesh=mesh,
        compiler_params=pltpu.CompilerParams(
            use_tc_tiling_on_sc=False, needs_layout_passes=False),
        scratch_types=[
            pltpu.VMEM((SUPER, CHUNK), jnp.int32),
            pltpu.VMEM((SUPER, CHUNK), jnp.int32),
            pltpu.VMEM((SUPER, CHUNK), jnp.int32),
            pltpu.VMEM((CBUF,), jnp.int32),
            pltpu.VMEM((CBUF,), jnp.int32),
            pltpu.VMEM((1, CHUNK), jnp.int32),
            pltpu.VMEM((CHUNK, HID), jnp.float32),
            pltpu.VMEM((CHUNK, HID), jnp.float32),
            pltpu.VMEM((OCH, HID), jnp.float32),
            pltpu.VMEM_SHARED((ACC_ROWS, HID), jnp.float32),
            pltpu.SemaphoreType.DMA,
            pltpu.SemaphoreType.DMA,
        ],
    )
    def sc_scatter(hl, tgt2, src2, typ2, zrows, out,
                   tgtbuf, srcbuf, typbuf, g1d, d1d, dwin, rows0, rows1,
                   obuf, acc, sem0, sem1):
        cid = lax.axis_index("c")
        sid = lax.axis_index("s")
        wid = sid * NC + cid

        zpt = ACC_ROWS // NS
        pltpu.sync_copy(zrows, obuf)
        for t in range(3):
            pltpu.sync_copy(obuf, acc.at[pl.ds(sid * zpt + t * OCH, OCH)])
        pltpu.sync_copy(obuf.at[pl.ds(0, zpt - 3 * OCH)],
                        acc.at[pl.ds(sid * zpt + 3 * OCH, zpt - 3 * OCH)])

        lanes = lax.iota(jnp.int32, 16)
        trash = TRASH0 + sid * 8 + (lanes & 7)

        def compact(s, off):
            row0 = wid * ROWS_PER_W + s * SUPER
            pltpu.sync_copy(tgt2.at[pl.ds(row0, SUPER)], tgtbuf)
            pltpu.sync_copy(src2.at[pl.ds(row0, SUPER)], srcbuf)
            pltpu.sync_copy(typ2.at[pl.ds(row0, SUPER)], typbuf)
            for r in range(SUPER):
                for k in range(CHUNK // 16):
                    sl = pl.ds(k * 16, 16)
                    m = typbuf[r, sl] == rel
                    plsc.store_compressed(g1d.at[pl.ds(off, 16)],
                                          tgtbuf[r, sl], mask=m)
                    plsc.store_compressed(d1d.at[pl.ds(off, 16)],
                                          srcbuf[r, sl], mask=m)
                    pc = plsc.all_reduce_population_count(m)
                    if pc.ndim:
                        pc = lax.squeeze(lax.slice(pc, (0,), (1,)), (0,))
                    off = off + pc
            return off

        cnt = lax.fori_loop(0, NSUPER, compact, jnp.int32(0))

        padg = (sid * 16 + lanes) * 8
        for k in range(2 * CHUNK // 16):
            g1d[pl.ds(cnt + k * 16, 16)] = padg
            d1d[pl.ds(cnt + k * 16, 16)] = trash
        npairs = lax.shift_right_logical(cnt + 255, 8)
        nchunks = npairs * 2

        plsc.subcore_barrier()

        def fire(c, buf, sem):
            return pltpu.async_copy(
                hl.at[g1d.at[pl.ds(c * CHUNK, CHUNK)]], buf, sem)

        def drain(buf, sem):
            pltpu.make_async_copy(
                hl.at[g1d.at[pl.ds(0, CHUNK)]], buf, sem).wait()

        def stage_scatter(c, buf):
            for k in range(CHUNK // 16):
                dwin[0, pl.ds(k * 16, 16)] = d1d[pl.ds(c * CHUNK + k * 16, 16)]
            pltpu.sync_copy(buf, acc.at[dwin.at[0]], add=True)

        fire(jnp.int32(0), rows0, sem0)

        def pair(i, carry):
            a = 2 * i
            fire(a + 1, rows1, sem1)
            drain(rows0, sem0)
            stage_scatter(a, rows0)
            fire(jnp.minimum(a + 2, nchunks - 1), rows0, sem0)
            drain(rows1, sem1)
            stage_scatter(a + 1, rows1)
            return carry

        lax.fori_loop(0, npairs, pair, 0)
        drain(rows0, sem0)

        plsc.subcore_barrier()
        for t in range(3):
            r0 = sid * (3 * OCH) + t * OCH
            pltpu.sync_copy(acc.at[pl.ds(r0, OCH)], obuf)
            pltpu.sync_copy(obuf, out.at[pl.ds(cid * N + r0, OCH)])

        @pl.when(sid == NS - 1)
        def _copy_tail():
            r0 = NS * 3 * OCH
            pltpu.sync_copy(acc.at[pl.ds(r0, N - r0)],
                            obuf.at[pl.ds(0, N - r0)])
            pltpu.sync_copy(obuf.at[pl.ds(0, N - r0)],
                            out.at[pl.ds(cid * N + r0, N - r0)])

    return sc_scatter


_sc_scatter = _make_sc_scatter()


def _tc_layer0(x, wcat, bias):
    def body(x_ref, w_ref, b_ref, hl_ref, d_ref):
        z = jnp.dot(x_ref[...], w_ref[...], preferred_element_type=jnp.float32)
        hl_ref[...] = z[:, :HID]
        d_ref[...] = z[:, HID:] + b_ref[...]

    return pl.pallas_call(
        body,
        grid=(N // BR,),
        in_specs=[
            pl.BlockSpec((BR, IN_CH), lambda i: (i, 0)),
            pl.BlockSpec((IN_CH, 2 * HID), lambda i: (0, 0)),
            pl.BlockSpec((1, HID), lambda i: (0, 0)),
        ],
        out_specs=[
            pl.BlockSpec((BR, HID), lambda i: (i, 0)),
            pl.BlockSpec((BR, HID), lambda i: (i, 0)),
        ],
        out_shape=[
            jax.ShapeDtypeStruct((N, HID), jnp.float32),
            jax.ShapeDtypeStruct((N, HID), jnp.float32),
        ],
    )(x, wcat, bias)


def _tc_mid(parts, dense, wcat, bias):
    def body(p_ref, d_ref, w_ref, b_ref, hl_ref, d1_ref):
        h1 = jnp.maximum(p_ref[0] + p_ref[1] + d_ref[...], 0.0)
        z = jnp.dot(h1, w_ref[...], preferred_element_type=jnp.float32)
        hl_ref[...] = z[:, :HID]
        d1_ref[...] = z[:, HID:] + b_ref[...]

    return pl.pallas_call(
        body,
        grid=(N // BR,),
        in_specs=[
            pl.BlockSpec((NC, BR, HID), lambda i: (0, i, 0)),
            pl.BlockSpec((BR, HID), lambda i: (i, 0)),
            pl.BlockSpec((HID, 2 * HID), lambda i: (0, 0)),
            pl.BlockSpec((1, HID), lambda i: (0, 0)),
        ],
        out_specs=[
            pl.BlockSpec((BR, HID), lambda i: (i, 0)),
            pl.BlockSpec((BR, HID), lambda i: (i, 0)),
        ],
        out_shape=[
            jax.ShapeDtypeStruct((N, HID), jnp.float32),
            jax.ShapeDtypeStruct((N, HID), jnp.float32),
        ],
    )(parts, dense, wcat, bias)


def _tc_final(parts, dense):
    def body(p_ref, d_ref, o_ref):
        o_ref[...] = jnp.maximum(p_ref[0] + p_ref[1] + d_ref[...], 0.0)

    return pl.pallas_call(
        body,
        grid=(N // BR,),
        in_specs=[
            pl.BlockSpec((NC, BR, HID), lambda i: (0, i, 0)),
            pl.BlockSpec((BR, HID), lambda i: (i, 0)),
        ],
        out_specs=pl.BlockSpec((BR, HID), lambda i: (i, 0)),
        out_shape=jax.ShapeDtypeStruct((N, HID), jnp.float32),
    )(parts, dense)


def kernel(x, edge_index, edge_type, w_l0, b_l0, w_00, b_00, w_10, b_10,
           w_l1, b_l1, w_01, b_01, w_11, b_11):
    src = edge_index[0]
    tgt = edge_index[1]

    pad = EPAD - E
    tgt_p = jnp.concatenate([tgt, jnp.arange(pad, dtype=jnp.int32) % N])
    src_p = jnp.concatenate([src, jnp.zeros((pad,), jnp.int32)])
    typ_p = jnp.concatenate([edge_type, jnp.full((pad,), 2, jnp.int32)])
    tgt2 = tgt_p.reshape(ROWS2D, CHUNK)
    src2 = src_p.reshape(ROWS2D, CHUNK)
    typ2 = typ_p.reshape(ROWS2D, CHUNK)
    zrows = jnp.zeros((OCH, HID), jnp.float32)

    wcat0 = jnp.concatenate([w_l0, w_00 + w_10], axis=1)
    bias0 = (b_l0 + b_00 + b_10)[None, :]
    wcat1 = jnp.concatenate([w_l1, w_01 + w_11], axis=1)
    bias1 = (b_l1 + b_01 + b_11)[None, :]

    typ2b = typ2 ^ 1

    hl0, dense0 = _tc_layer0(x, wcat0, bias0)
    parts0 = _sc_scatter(hl0, tgt2, src2, typ2, zrows).reshape(NC, N, HID)
    hl1, dense1 = _tc_mid(parts0, dense0, wcat1, bias1)
    parts1 = _sc_scatter(hl1, tgt2, src2, typ2b, zrows).reshape(NC, N, HID)
    return _tc_final(parts1, dense1)


# speedup vs baseline: 27.9185x; 1.0001x over previous
import functools

import jax
import jax.numpy as jnp
from jax import lax
from jax.experimental import pallas as pl
from jax.experimental.pallas import tpu as pltpu
from jax.experimental.pallas import tpu_sc as plsc

N = 10000
E = 640000
IN_CH = 128
HID = 64

NC = 2
NS = 16
NW = NC * NS
CHUNK = 128
SUPER = 16
NSUPER = 10
EPAD = NW * NSUPER * SUPER * CHUNK
ROWS2D = EPAD // CHUNK
ROWS_PER_W = NSUPER * SUPER

ACC_ROWS = 10144
TRASH0 = 10016
OCH = 208

EDGES_PER_W = ROWS_PER_W * CHUNK
CBUF = EDGES_PER_W + 2 * CHUNK

BR = 1000


def _make_sc_scatter():
    rel = 0
    mesh = plsc.VectorSubcoreMesh(
        core_axis_name="c", subcore_axis_name="s", num_cores=NC, num_subcores=NS
    )

    @functools.partial(
        pl.kernel,
        out_type=jax.ShapeDtypeStruct((NC * N, HID), jnp.float32),
        mesh=mesh,
        compiler_params=pltpu.CompilerParams(
            use_tc_tiling_on_sc=False, needs_layout_passes=False),
        scratch_types=[
            pltpu.VMEM((SUPER, CHUNK), jnp.int32),
            pltpu.VMEM((SUPER, CHUNK), jnp.int32),
            pltpu.VMEM((SUPER, CHUNK), jnp.int32),
            pltpu.VMEM((CBUF,), jnp.int32),
            pltpu.VMEM((CBUF,), jnp.int32),
            pltpu.VMEM((1, CHUNK), jnp.int32),
            pltpu.VMEM((CHUNK, HID), jnp.float32),
            pltpu.VMEM((CHUNK, HID), jnp.float32),
            pltpu.VMEM((OCH, HID), jnp.float32),
            pltpu.VMEM_SHARED((ACC_ROWS, HID), jnp.float32),
            pltpu.SemaphoreType.DMA,
            pltpu.SemaphoreType.DMA,
        ],
    )
    def sc_scatter(hl, tgt2, src2, typ2, zrows, out,
                   tgtbuf, srcbuf, typbuf, g1d, d1d, dwin, rows0, rows1,
                   obuf, acc, sem0, sem1):
        cid = lax.axis_index("c")
        sid = lax.axis_index("s")
        wid = sid * NC + cid

        zpt = ACC_ROWS // NS
        pltpu.sync_copy(zrows, obuf)
        for t in range(3):
            pltpu.sync_copy(obuf, acc.at[pl.ds(sid * zpt + t * OCH, OCH)])
        pltpu.sync_copy(obuf.at[pl.ds(0, zpt - 3 * OCH)],
                        acc.at[pl.ds(sid * zpt + 3 * OCH, zpt - 3 * OCH)])

        lanes = lax.iota(jnp.int32, 16)
        trash = TRASH0 + sid * 8 + (lanes & 7)

        def compact(s, off):
            row0 = wid * ROWS_PER_W + s * SUPER
            pltpu.sync_copy(tgt2.at[pl.ds(row0, SUPER)], tgtbuf)
            pltpu.sync_copy(src2.at[pl.ds(row0, SUPER)], srcbuf)
            pltpu.sync_copy(typ2.at[pl.ds(row0, SUPER)], typbuf)
            for r in range(SUPER):
                for k in range(CHUNK // 16):
                    sl = pl.ds(k * 16, 16)
                    m = typbuf[r, sl] == rel
                    plsc.store_compressed(g1d.at[pl.ds(off, 16)],
                                          tgtbuf[r, sl], mask=m)
                    plsc.store_compressed(d1d.at[pl.ds(off, 16)],
                                          srcbuf[r, sl], mask=m)
                    pc = plsc.all_reduce_population_count(m)
                    if pc.ndim:
                        pc = lax.squeeze(lax.slice(pc, (0,), (1,)), (0,))
                    off = off + pc
            return off

        cnt = lax.fori_loop(0, NSUPER, compact, jnp.int32(0))

        padg = (sid * 16 + lanes) * 8
        for k in range(2 * CHUNK // 16):
            g1d[pl.ds(cnt + k * 16, 16)] = padg
            d1d[pl.ds(cnt + k * 16, 16)] = trash
        npairs = lax.shift_right_logical(cnt + 255, 8)
        nchunks = npairs * 2

        plsc.subcore_barrier()

        def fire(c, buf, sem):
            return pltpu.async_copy(
                hl.at[g1d.at[pl.ds(c * CHUNK, CHUNK)]], buf, sem)

        def drain(buf, sem):
            pltpu.make_async_copy(
                hl.at[g1d.at[pl.ds(0, CHUNK)]], buf, sem).wait()

        def stage_scatter(c, buf):
            for k in range(CHUNK // 16):
                dwin[0, pl.ds(k * 16, 16)] = d1d[pl.ds(c * CHUNK + k * 16, 16)]
            pltpu.sync_copy(buf, acc.at[dwin.at[0]], add=True)

        fire(jnp.int32(0), rows0, sem0)

        def pair(i, carry):
            a = 2 * i
            fire(a + 1, rows1, sem1)
            drain(rows0, sem0)
            stage_scatter(a, rows0)
            fire(jnp.minimum(a + 2, nchunks - 1), rows0, sem0)
            drain(rows1, sem1)
            stage_scatter(a + 1, rows1)
            return carry

        lax.fori_loop(0, npairs, pair, 0)
        drain(rows0, sem0)

        plsc.subcore_barrier()
        for t in range(3):
            r0 = sid * (3 * OCH) + t * OCH
            pltpu.sync_copy(acc.at[pl.ds(r0, OCH)], obuf)
            pltpu.sync_copy(obuf, out.at[pl.ds(cid * N + r0, OCH)])

        @pl.when(sid == NS - 1)
        def _copy_tail():
            r0 = NS * 3 * OCH
            pltpu.sync_copy(acc.at[pl.ds(r0, N - r0)],
                            obuf.at[pl.ds(0, N - r0)])
            pltpu.sync_copy(obuf.at[pl.ds(0, N - r0)],
                            out.at[pl.ds(cid * N + r0, N - r0)])

    return sc_scatter


_sc_scatter = _make_sc_scatter()


def _tc_layer0(x, wcat, bias):
    def body(x_ref, w_ref, b_ref, hl_ref, d_ref):
        z = jnp.dot(x_ref[...], w_ref[...], preferred_element_type=jnp.float32)
        hl_ref[...] = z[:, :HID]
        d_ref[...] = z[:, HID:] + b_ref[...]

    return pl.pallas_call(
        body,
        grid=(N // BR,),
        in_specs=[
            pl.BlockSpec((BR, IN_CH), lambda i: (i, 0)),
            pl.BlockSpec((IN_CH, 2 * HID), lambda i: (0, 0)),
            pl.BlockSpec((1, HID), lambda i: (0, 0)),
        ],
        out_specs=[
            pl.BlockSpec((BR, HID), lambda i: (i, 0)),
            pl.BlockSpec((BR, HID), lambda i: (i, 0)),
        ],
        out_shape=[
            jax.ShapeDtypeStruct((N, HID), jnp.float32),
            jax.ShapeDtypeStruct((N, HID), jnp.float32),
        ],
    )(x, wcat, bias)


def _tc_mid(parts, dense, wcat, bias):
    def body(p0_ref, p1_ref, d_ref, w_ref, b_ref, hl_ref, d1_ref):
        h1 = jnp.maximum(p0_ref[...] + p1_ref[...] + d_ref[...], 0.0)
        z = jnp.dot(h1, w_ref[...], preferred_element_type=jnp.float32)
        hl_ref[...] = z[:, :HID]
        d1_ref[...] = z[:, HID:] + b_ref[...]

    return pl.pallas_call(
        body,
        grid=(N // BR,),
        in_specs=[
            pl.BlockSpec((BR, HID), lambda i: (i, 0)),
            pl.BlockSpec((BR, HID), lambda i: (i + N // BR, 0)),
            pl.BlockSpec((BR, HID), lambda i: (i, 0)),
            pl.BlockSpec((HID, 2 * HID), lambda i: (0, 0)),
            pl.BlockSpec((1, HID), lambda i: (0, 0)),
        ],
        out_specs=[
            pl.BlockSpec((BR, HID), lambda i: (i, 0)),
            pl.BlockSpec((BR, HID), lambda i: (i, 0)),
        ],
        out_shape=[
            jax.ShapeDtypeStruct((N, HID), jnp.float32),
            jax.ShapeDtypeStruct((N, HID), jnp.float32),
        ],
    )(parts, parts, dense, wcat, bias)


def _tc_final(parts, dense):
    def body(p0_ref, p1_ref, d_ref, o_ref):
        o_ref[...] = jnp.maximum(p0_ref[...] + p1_ref[...] + d_ref[...], 0.0)

    return pl.pallas_call(
        body,
        grid=(N // BR,),
        in_specs=[
            pl.BlockSpec((BR, HID), lambda i: (i, 0)),
            pl.BlockSpec((BR, HID), lambda i: (i + N // BR, 0)),
            pl.BlockSpec((BR, HID), lambda i: (i, 0)),
        ],
        out_specs=pl.BlockSpec((BR, HID), lambda i: (i, 0)),
        out_shape=jax.ShapeDtypeStruct((N, HID), jnp.float32),
    )(parts, parts, dense)


def kernel(x, edge_index, edge_type, w_l0, b_l0, w_00, b_00, w_10, b_10,
           w_l1, b_l1, w_01, b_01, w_11, b_11):
    src = edge_index[0]
    tgt = edge_index[1]

    pad = EPAD - E
    tgt_p = jnp.concatenate([tgt, jnp.arange(pad, dtype=jnp.int32) % N])
    src_p = jnp.concatenate([src, jnp.zeros((pad,), jnp.int32)])
    typ_p = jnp.concatenate([edge_type, jnp.full((pad,), 2, jnp.int32)])
    tgt2 = tgt_p.reshape(ROWS2D, CHUNK)
    src2 = src_p.reshape(ROWS2D, CHUNK)
    typ2 = typ_p.reshape(ROWS2D, CHUNK)
    zrows = jnp.zeros((OCH, HID), jnp.float32)

    wcat0 = jnp.concatenate([w_l0, w_00 + w_10], axis=1)
    bias0 = (b_l0 + b_00 + b_10)[None, :]
    wcat1 = jnp.concatenate([w_l1, w_01 + w_11], axis=1)
    bias1 = (b_l1 + b_01 + b_11)[None, :]

    typ2b = typ2 ^ 1

    hl0, dense0 = _tc_layer0(x, wcat0, bias0)
    parts0 = _sc_scatter(hl0, tgt2, src2, typ2, zrows)
    hl1, dense1 = _tc_mid(parts0, dense0, wcat1, bias1)
    parts1 = _sc_scatter(hl1, tgt2, src2, typ2b, zrows)
    return _tc_final(parts1, dense1)


# speedup vs baseline: 31.1260x; 1.1149x over previous
import functools

import jax
import jax.numpy as jnp
from jax import lax
from jax.experimental import pallas as pl
from jax.experimental.pallas import tpu as pltpu
from jax.experimental.pallas import tpu_sc as plsc

N = 10000
E = 640000
IN_CH = 128
HID = 64

NC = 2
NS = 16
NW = NC * NS
CHUNK = 128
SUPER = 16
NSUPER = 10
EPAD = NW * NSUPER * SUPER * CHUNK
ROWS2D = EPAD // CHUNK
ROWS_PER_W = NSUPER * SUPER

ACC_ROWS = 10144
TRASH0 = 10016
OCH = 208

EDGES_PER_W = ROWS_PER_W * CHUNK
CBUF = EDGES_PER_W + 2 * CHUNK

BR = 1000


def _make_sc_scatter():
    rel = 0
    mesh = plsc.VectorSubcoreMesh(
        core_axis_name="c", subcore_axis_name="s", num_cores=NC, num_subcores=NS
    )

    @functools.partial(
        pl.kernel,
        out_type=jax.ShapeDtypeStruct((N, NC * HID), jnp.float32),
        mesh=mesh,
        compiler_params=pltpu.CompilerParams(
            use_tc_tiling_on_sc=False, needs_layout_passes=False),
        scratch_types=[
            pltpu.VMEM((SUPER, CHUNK), jnp.int32),
            pltpu.VMEM((SUPER, CHUNK), jnp.int32),
            pltpu.VMEM((SUPER, CHUNK), jnp.int32),
            pltpu.VMEM((CBUF,), jnp.int32),
            pltpu.VMEM((CBUF,), jnp.int32),
            pltpu.VMEM((1, CHUNK), jnp.int32),
            pltpu.VMEM((CHUNK, HID), jnp.float32),
            pltpu.VMEM((CHUNK, HID), jnp.float32),
            pltpu.VMEM((OCH, HID), jnp.float32),
            pltpu.VMEM_SHARED((ACC_ROWS, HID), jnp.float32),
            pltpu.SemaphoreType.DMA,
            pltpu.SemaphoreType.DMA,
        ],
    )
    def sc_scatter(hl, tgt2, src2, typ2, zrows, out,
                   tgtbuf, srcbuf, typbuf, g1d, d1d, dwin, rows0, rows1,
                   obuf, acc, sem0, sem1):
        cid = lax.axis_index("c")
        sid = lax.axis_index("s")
        wid = sid * NC + cid

        zpt = ACC_ROWS // NS
        pltpu.sync_copy(zrows, obuf)
        for t in range(3):
            pltpu.sync_copy(obuf, acc.at[pl.ds(sid * zpt + t * OCH, OCH)])
        pltpu.sync_copy(obuf.at[pl.ds(0, zpt - 3 * OCH)],
                        acc.at[pl.ds(sid * zpt + 3 * OCH, zpt - 3 * OCH)])

        lanes = lax.iota(jnp.int32, 16)
        trash = TRASH0 + sid * 8 + (lanes & 7)

        def compact(s, off):
            row0 = wid * ROWS_PER_W + s * SUPER
            pltpu.sync_copy(tgt2.at[pl.ds(row0, SUPER)], tgtbuf)
            pltpu.sync_copy(src2.at[pl.ds(row0, SUPER)], srcbuf)
            pltpu.sync_copy(typ2.at[pl.ds(row0, SUPER)], typbuf)
            for r in range(SUPER):
                for k in range(CHUNK // 16):
                    sl = pl.ds(k * 16, 16)
                    m = typbuf[r, sl] == rel
                    tv = tgtbuf[r, sl]
                    plsc.store_compressed(g1d.at[pl.ds(off, 16)],
                                          tv + tv, mask=m)
                    plsc.store_compressed(d1d.at[pl.ds(off, 16)],
                                          srcbuf[r, sl], mask=m)
                    pc = plsc.all_reduce_population_count(m)
                    if pc.ndim:
                        pc = lax.squeeze(lax.slice(pc, (0,), (1,)), (0,))
                    off = off + pc
            return off

        cnt = lax.fori_loop(0, NSUPER, compact, jnp.int32(0))

        padg = (sid * 16 + lanes) * 8
        for k in range(2 * CHUNK // 16):
            g1d[pl.ds(cnt + k * 16, 16)] = padg
            d1d[pl.ds(cnt + k * 16, 16)] = trash
        npairs = lax.shift_right_logical(cnt + 255, 8)
        nchunks = npairs * 2

        plsc.subcore_barrier()

        def fire(c, buf, sem):
            return pltpu.async_copy(
                hl.at[g1d.at[pl.ds(c * CHUNK, CHUNK)]], buf, sem)

        def drain(buf, sem):
            pltpu.make_async_copy(
                hl.at[g1d.at[pl.ds(0, CHUNK)]], buf, sem).wait()

        def stage_scatter(c, buf):
            for k in range(CHUNK // 16):
                dwin[0, pl.ds(k * 16, 16)] = d1d[pl.ds(c * CHUNK + k * 16, 16)]
            pltpu.sync_copy(buf, acc.at[dwin.at[0]], add=True)

        fire(jnp.int32(0), rows0, sem0)

        def pair(i, carry):
            a = 2 * i
            fire(a + 1, rows1, sem1)
            drain(rows0, sem0)
            stage_scatter(a, rows0)
            fire(jnp.minimum(a + 2, nchunks - 1), rows0, sem0)
            drain(rows1, sem1)
            stage_scatter(a + 1, rows1)
            return carry

        lax.fori_loop(0, npairs, pair, 0)
        drain(rows0, sem0)

        plsc.subcore_barrier()
        cslice = pl.ds(cid * HID, HID)
        for t in range(3):
            r0 = sid * (3 * OCH) + t * OCH
            pltpu.sync_copy(acc.at[pl.ds(r0, OCH)], obuf)
            pltpu.sync_copy(obuf, out.at[pl.ds(r0, OCH), cslice])

        @pl.when(sid == NS - 1)
        def _copy_tail():
            r0 = NS * 3 * OCH
            pltpu.sync_copy(acc.at[pl.ds(r0, N - r0)],
                            obuf.at[pl.ds(0, N - r0)])
            pltpu.sync_copy(obuf.at[pl.ds(0, N - r0)],
                            out.at[pl.ds(r0, N - r0), cslice])

    return sc_scatter


_sc_scatter = _make_sc_scatter()


def _tc_layer0(x, wcat, bias_full):
    def body(x_ref, w_ref, b_ref, z_ref):
        z_ref[...] = jnp.dot(
            x_ref[...], w_ref[...], preferred_element_type=jnp.float32
        ) + b_ref[...]

    return pl.pallas_call(
        body,
        grid=(N // BR,),
        in_specs=[
            pl.BlockSpec((BR, IN_CH), lambda i: (i, 0)),
            pl.BlockSpec((IN_CH, 2 * HID), lambda i: (0, 0)),
            pl.BlockSpec((1, 2 * HID), lambda i: (0, 0)),
        ],
        out_specs=pl.BlockSpec((BR, 2 * HID), lambda i: (i, 0)),
        out_shape=jax.ShapeDtypeStruct((N, 2 * HID), jnp.float32),
    )(x, wcat, bias_full)


def _tc_mid(parts, zin, wcat, bias_full):
    def body(p_ref, z_ref, w_ref, b_ref, z1_ref):
        h1 = jnp.maximum(p_ref[:, :HID] + p_ref[:, HID:] + z_ref[:, HID:], 0.0)
        z1_ref[...] = jnp.dot(
            h1, w_ref[...], preferred_element_type=jnp.float32
        ) + b_ref[...]

    return pl.pallas_call(
        body,
        grid=(N // BR,),
        in_specs=[
            pl.BlockSpec((BR, 2 * HID), lambda i: (i, 0)),
            pl.BlockSpec((BR, 2 * HID), lambda i: (i, 0)),
            pl.BlockSpec((HID, 2 * HID), lambda i: (0, 0)),
            pl.BlockSpec((1, 2 * HID), lambda i: (0, 0)),
        ],
        out_specs=pl.BlockSpec((BR, 2 * HID), lambda i: (i, 0)),
        out_shape=jax.ShapeDtypeStruct((N, 2 * HID), jnp.float32),
    )(parts, zin, wcat, bias_full)


def _tc_final(parts, zin):
    def body(p_ref, z_ref, o_ref):
        o_ref[...] = jnp.maximum(
            p_ref[:, :HID] + p_ref[:, HID:] + z_ref[:, HID:], 0.0)

    return pl.pallas_call(
        body,
        grid=(N // BR,),
        in_specs=[
            pl.BlockSpec((BR, 2 * HID), lambda i: (i, 0)),
            pl.BlockSpec((BR, 2 * HID), lambda i: (i, 0)),
        ],
        out_specs=pl.BlockSpec((BR, HID), lambda i: (i, 0)),
        out_shape=jax.ShapeDtypeStruct((N, HID), jnp.float32),
    )(parts, zin)


def kernel(x, edge_index, edge_type, w_l0, b_l0, w_00, b_00, w_10, b_10,
           w_l1, b_l1, w_01, b_01, w_11, b_11):
    src = edge_index[0]
    tgt = edge_index[1]

    pad = EPAD - E
    tgt_p = jnp.concatenate([tgt, jnp.arange(pad, dtype=jnp.int32) % N])
    src_p = jnp.concatenate([src, jnp.zeros((pad,), jnp.int32)])
    typ_p = jnp.concatenate([edge_type, jnp.full((pad,), 2, jnp.int32)])
    tgt2 = tgt_p.reshape(ROWS2D, CHUNK)
    src2 = src_p.reshape(ROWS2D, CHUNK)
    typ2 = typ_p.reshape(ROWS2D, CHUNK)
    zrows = jnp.zeros((OCH, HID), jnp.float32)

    zeros_h = jnp.zeros((HID,), jnp.float32)
    wcat0 = jnp.concatenate([w_l0, w_00 + w_10], axis=1)
    bias0 = jnp.concatenate([zeros_h, b_l0 + b_00 + b_10])[None, :]
    wcat1 = jnp.concatenate([w_l1, w_01 + w_11], axis=1)
    bias1 = jnp.concatenate([zeros_h, b_l1 + b_01 + b_11])[None, :]

    typ2b = typ2 ^ 1

    z0 = _tc_layer0(x, wcat0, bias0)
    parts0 = _sc_scatter(z0.reshape(2 * N, HID), tgt2, src2, typ2, zrows)
    z1 = _tc_mid(parts0, z0, wcat1, bias1)
    parts1 = _sc_scatter(z1.reshape(2 * N, HID), tgt2, src2, typ2b, zrows)
    return _tc_final(parts1, z1)


# speedup vs baseline: 34.5968x; 1.1115x over previous
import functools

import jax
import jax.numpy as jnp
from jax import lax
from jax.experimental import pallas as pl
from jax.experimental.pallas import tpu as pltpu
from jax.experimental.pallas import tpu_sc as plsc

N = 10000
E = 640000
IN_CH = 128
HID = 64

NC = 2
NS = 16
NW = NC * NS
CHUNK = 128
SUPER = 16
NSUPER = 10
EPAD = NW * NSUPER * SUPER * CHUNK
ROWS2D = EPAD // CHUNK
ROWS_PER_W = NSUPER * SUPER

ACC_ROWS = 10144
TRASH0 = 10016
OCH = 208

EDGES_PER_W = ROWS_PER_W * CHUNK
CBUF = EDGES_PER_W + 4 * CHUNK

BR = 2000


def _make_sc_scatter():
    rel = 0
    mesh = plsc.VectorSubcoreMesh(
        core_axis_name="c", subcore_axis_name="s", num_cores=NC, num_subcores=NS
    )

    @functools.partial(
        pl.kernel,
        out_type=jax.ShapeDtypeStruct((N, NC * HID), jnp.float32),
        mesh=mesh,
        compiler_params=pltpu.CompilerParams(
            use_tc_tiling_on_sc=False, needs_layout_passes=False),
        scratch_types=[
            pltpu.VMEM((SUPER, CHUNK), jnp.int32),
            pltpu.VMEM((SUPER, CHUNK), jnp.int32),
            pltpu.VMEM((SUPER, CHUNK), jnp.int32),
            pltpu.VMEM((CBUF,), jnp.int32),
            pltpu.VMEM((CBUF,), jnp.int32),
            pltpu.VMEM((3, CHUNK), jnp.int32),
            pltpu.VMEM((CHUNK, HID), jnp.float32),
            pltpu.VMEM((CHUNK, HID), jnp.float32),
            pltpu.VMEM((CHUNK, HID), jnp.float32),
            pltpu.VMEM((OCH, HID), jnp.float32),
            pltpu.VMEM_SHARED((ACC_ROWS, HID), jnp.float32),
            pltpu.SemaphoreType.DMA,
            pltpu.SemaphoreType.DMA,
            pltpu.SemaphoreType.DMA,
            pltpu.SemaphoreType.DMA,
            pltpu.SemaphoreType.DMA,
            pltpu.SemaphoreType.DMA,
        ],
    )
    def sc_scatter(hl, tgt2, src2, typ2, zrows, out,
                   tgtbuf, srcbuf, typbuf, g1d, d1d, dwin, rows0, rows1, rows2,
                   obuf, acc, gsem0, gsem1, gsem2, ssem0, ssem1, ssem2):
        cid = lax.axis_index("c")
        sid = lax.axis_index("s")
        wid = sid * NC + cid

        zpt = ACC_ROWS // NS
        pltpu.sync_copy(zrows, obuf)
        for t in range(3):
            pltpu.sync_copy(obuf, acc.at[pl.ds(sid * zpt + t * OCH, OCH)])
        pltpu.sync_copy(obuf.at[pl.ds(0, zpt - 3 * OCH)],
                        acc.at[pl.ds(sid * zpt + 3 * OCH, zpt - 3 * OCH)])

        lanes = lax.iota(jnp.int32, 16)
        trash = TRASH0 + sid * 8 + (lanes & 7)

        def compact(s, off):
            row0 = wid * ROWS_PER_W + s * SUPER
            pltpu.sync_copy(tgt2.at[pl.ds(row0, SUPER)], tgtbuf)
            pltpu.sync_copy(src2.at[pl.ds(row0, SUPER)], srcbuf)
            pltpu.sync_copy(typ2.at[pl.ds(row0, SUPER)], typbuf)
            for r in range(SUPER):
                for k in range(CHUNK // 16):
                    sl = pl.ds(k * 16, 16)
                    m = typbuf[r, sl] == rel
                    tv = tgtbuf[r, sl]
                    plsc.store_compressed(g1d.at[pl.ds(off, 16)],
                                          tv + tv, mask=m)
                    plsc.store_compressed(d1d.at[pl.ds(off, 16)],
                                          srcbuf[r, sl], mask=m)
                    pc = plsc.all_reduce_population_count(m)
                    if pc.ndim:
                        pc = lax.squeeze(lax.slice(pc, (0,), (1,)), (0,))
                    off = off + pc
            return off

        cnt = lax.fori_loop(0, NSUPER, compact, jnp.int32(0))

        padg = (sid * 16 + lanes) * 8
        for k in range(3 * CHUNK // 16):
            g1d[pl.ds(cnt + k * 16, 16)] = padg
            d1d[pl.ds(cnt + k * 16, 16)] = trash
        ntri = (cnt + 383) // jnp.int32(384)
        nchunks = ntri * 3

        plsc.subcore_barrier()

        rows = (rows0, rows1, rows2)
        gsems = (gsem0, gsem1, gsem2)
        ssems = (ssem0, ssem1, ssem2)

        def fire(c, b):
            return pltpu.async_copy(
                hl.at[g1d.at[pl.ds(c * CHUNK, CHUNK)]], rows[b], gsems[b])

        def gdrain(b):
            pltpu.make_async_copy(
                hl.at[g1d.at[pl.ds(0, CHUNK)]], rows[b], gsems[b]).wait()

        def swait(b):
            pltpu.make_async_copy(
                rows[b], acc.at[dwin.at[b]], ssems[b]).wait()

        def chunk_step(i, u, first):
            c = 3 * i + u
            gdrain(u)
            for k in range(CHUNK // 16):
                dwin[u, pl.ds(k * 16, 16)] = d1d[pl.ds(c * CHUNK + k * 16, 16)]
            pltpu.async_copy(rows[u], acc.at[dwin.at[u]], ssems[u], add=True)
            nb = (u + 2) % 3
            if first:
                fire(c + 2, nb)
            else:
                swait(nb)
                fire(jnp.minimum(c + 2, nchunks - 1), nb)

        fire(jnp.int32(0), 0)
        fire(jnp.int32(1), 1)

        @pl.when(ntri > 0)
        def _first_triple():
            chunk_step(jnp.int32(0), 0, True)
            chunk_step(jnp.int32(0), 1, False)
            chunk_step(jnp.int32(0), 2, False)

        def triple(i, carry):
            chunk_step(i, 0, False)
            chunk_step(i, 1, False)
            chunk_step(i, 2, False)
            return carry

        lax.fori_loop(1, ntri, triple, 0)

        gdrain(0)
        gdrain(1)

        @pl.when(ntri > 0)
        def _drain_tail():
            swait(2)

        plsc.subcore_barrier()
        cslice = pl.ds(cid * HID, HID)
        for t in range(3):
            r0 = sid * (3 * OCH) + t * OCH
            pltpu.sync_copy(acc.at[pl.ds(r0, OCH)], obuf)
            pltpu.sync_copy(obuf, out.at[pl.ds(r0, OCH), cslice])

        @pl.when(sid == NS - 1)
        def _copy_tail():
            r0 = NS * 3 * OCH
            pltpu.sync_copy(acc.at[pl.ds(r0, N - r0)],
                            obuf.at[pl.ds(0, N - r0)])
            pltpu.sync_copy(obuf.at[pl.ds(0, N - r0)],
                            out.at[pl.ds(r0, N - r0), cslice])

    return sc_scatter


_sc_scatter = _make_sc_scatter()


def _tc_layer0(x, wcat, bias_full):
    def body(x_ref, w_ref, b_ref, z_ref):
        z_ref[...] = jnp.dot(
            x_ref[...], w_ref[...], preferred_element_type=jnp.float32
        ) + b_ref[...]

    return pl.pallas_call(
        body,
        grid=(N // BR,),
        in_specs=[
            pl.BlockSpec((BR, IN_CH), lambda i: (i, 0)),
            pl.BlockSpec((IN_CH, 2 * HID), lambda i: (0, 0)),
            pl.BlockSpec((1, 2 * HID), lambda i: (0, 0)),
        ],
        out_specs=pl.BlockSpec((BR, 2 * HID), lambda i: (i, 0)),
        out_shape=jax.ShapeDtypeStruct((N, 2 * HID), jnp.float32),
    )(x, wcat, bias_full)


def _tc_mid(parts, zin, wcat, bias_full):
    def body(p_ref, z_ref, w_ref, b_ref, z1_ref):
        h1 = jnp.maximum(p_ref[:, :HID] + p_ref[:, HID:] + z_ref[:, HID:], 0.0)
        z1_ref[...] = jnp.dot(
            h1, w_ref[...], preferred_element_type=jnp.float32
        ) + b_ref[...]

    return pl.pallas_call(
        body,
        grid=(N // BR,),
        in_specs=[
            pl.BlockSpec((BR, 2 * HID), lambda i: (i, 0)),
            pl.BlockSpec((BR, 2 * HID), lambda i: (i, 0)),
            pl.BlockSpec((HID, 2 * HID), lambda i: (0, 0)),
            pl.BlockSpec((1, 2 * HID), lambda i: (0, 0)),
        ],
        out_specs=pl.BlockSpec((BR, 2 * HID), lambda i: (i, 0)),
        out_shape=jax.ShapeDtypeStruct((N, 2 * HID), jnp.float32),
    )(parts, zin, wcat, bias_full)


def _tc_final(parts, zin):
    def body(p_ref, z_ref, o_ref):
        o_ref[...] = jnp.maximum(
            p_ref[:, :HID] + p_ref[:, HID:] + z_ref[:, HID:], 0.0)

    return pl.pallas_call(
        body,
        grid=(N // BR,),
        in_specs=[
            pl.BlockSpec((BR, 2 * HID), lambda i: (i, 0)),
            pl.BlockSpec((BR, 2 * HID), lambda i: (i, 0)),
        ],
        out_specs=pl.BlockSpec((BR, HID), lambda i: (i, 0)),
        out_shape=jax.ShapeDtypeStruct((N, HID), jnp.float32),
    )(parts, zin)


def kernel(x, edge_index, edge_type, w_l0, b_l0, w_00, b_00, w_10, b_10,
           w_l1, b_l1, w_01, b_01, w_11, b_11):
    src = edge_index[0]
    tgt = edge_index[1]

    pad = EPAD - E
    tgt_p = jnp.concatenate([tgt, jnp.arange(pad, dtype=jnp.int32) % N])
    src_p = jnp.concatenate([src, jnp.zeros((pad,), jnp.int32)])
    typ_p = jnp.concatenate([edge_type, jnp.full((pad,), 2, jnp.int32)])
    tgt2 = tgt_p.reshape(ROWS2D, CHUNK)
    src2 = src_p.reshape(ROWS2D, CHUNK)
    typ2 = typ_p.reshape(ROWS2D, CHUNK)
    zrows = jnp.zeros((OCH, HID), jnp.float32)

    zeros_h = jnp.zeros((HID,), jnp.float32)
    wcat0 = jnp.concatenate([w_l0, w_00 + w_10], axis=1)
    bias0 = jnp.concatenate([zeros_h, b_l0 + b_00 + b_10])[None, :]
    wcat1 = jnp.concatenate([w_l1, w_01 + w_11], axis=1)
    bias1 = jnp.concatenate([zeros_h, b_l1 + b_01 + b_11])[None, :]

    typ2b = typ2 ^ 1

    z0 = _tc_layer0(x, wcat0, bias0)
    parts0 = _sc_scatter(z0.reshape(2 * N, HID), tgt2, src2, typ2, zrows)
    z1 = _tc_mid(parts0, z0, wcat1, bias1)
    parts1 = _sc_scatter(z1.reshape(2 * N, HID), tgt2, src2, typ2b, zrows)
    return _tc_final(parts1, z1)


# speedup vs baseline: 34.6417x; 1.0013x over previous
import functools

import jax
import jax.numpy as jnp
from jax import lax
from jax.experimental import pallas as pl
from jax.experimental.pallas import tpu as pltpu
from jax.experimental.pallas import tpu_sc as plsc

N = 10000
E = 640000
IN_CH = 128
HID = 64

NC = 2
NS = 16
NW = NC * NS
CHUNK = 128
SUPER = 16
NSUPER = 10
EROWS = E // CHUNK
RPW = EROWS // NW
RREM = EROWS - NW * RPW

ACC_ROWS = 10144
TRASH0 = 10016
OCH = 208

EDGES_PER_W = (RPW + 1) * CHUNK
CBUF = EDGES_PER_W + 4 * CHUNK

BR = 2000


def _make_sc_scatter():
    rel = 0
    mesh = plsc.VectorSubcoreMesh(
        core_axis_name="c", subcore_axis_name="s", num_cores=NC, num_subcores=NS
    )

    @functools.partial(
        pl.kernel,
        out_type=jax.ShapeDtypeStruct((N, NC * HID), jnp.float32),
        mesh=mesh,
        compiler_params=pltpu.CompilerParams(
            use_tc_tiling_on_sc=False, needs_layout_passes=False),
        scratch_types=[
            pltpu.VMEM((SUPER, CHUNK), jnp.int32),
            pltpu.VMEM((SUPER, CHUNK), jnp.int32),
            pltpu.VMEM((SUPER, CHUNK), jnp.int32),
            pltpu.VMEM((CBUF,), jnp.int32),
            pltpu.VMEM((CBUF,), jnp.int32),
            pltpu.VMEM((3, CHUNK), jnp.int32),
            pltpu.VMEM((CHUNK, HID), jnp.float32),
            pltpu.VMEM((CHUNK, HID), jnp.float32),
            pltpu.VMEM((CHUNK, HID), jnp.float32),
            pltpu.VMEM((OCH, HID), jnp.float32),
            pltpu.VMEM_SHARED((ACC_ROWS, HID), jnp.float32),
            pltpu.SemaphoreType.DMA,
            pltpu.SemaphoreType.DMA,
            pltpu.SemaphoreType.DMA,
            pltpu.SemaphoreType.DMA,
            pltpu.SemaphoreType.DMA,
            pltpu.SemaphoreType.DMA,
        ],
    )
    def sc_scatter(hl, eidx2, typ2, zrows, out,
                   tgtbuf, srcbuf, typbuf, g1d, d1d, dwin, rows0, rows1, rows2,
                   obuf, acc, gsem0, gsem1, gsem2, ssem0, ssem1, ssem2):
        cid = lax.axis_index("c")
        sid = lax.axis_index("s")
        wid = sid * NC + cid

        zpt = ACC_ROWS // NS
        pltpu.sync_copy(zrows, obuf)
        for t in range(3):
            pltpu.sync_copy(obuf, acc.at[pl.ds(sid * zpt + t * OCH, OCH)])
        pltpu.sync_copy(obuf.at[pl.ds(0, zpt - 3 * OCH)],
                        acc.at[pl.ds(sid * zpt + 3 * OCH, zpt - 3 * OCH)])

        lanes = lax.iota(jnp.int32, 16)
        trash = TRASH0 + sid * 8 + (lanes & 7)

        c0 = wid * RPW + jnp.minimum(wid, RREM)
        nw = RPW + jnp.where(wid < RREM, 1, 0)

        def compact(s, off):
            base = jnp.minimum(c0 + SUPER * s, c0 + nw - SUPER)
            lim = c0 + SUPER * s
            pltpu.sync_copy(eidx2.at[pl.ds(EROWS + base, SUPER)], tgtbuf)
            pltpu.sync_copy(eidx2.at[pl.ds(base, SUPER)], srcbuf)
            pltpu.sync_copy(typ2.at[pl.ds(base, SUPER)], typbuf)
            for r in range(SUPER):
                rvalid = base + r >= lim
                for k in range(CHUNK // 16):
                    sl = pl.ds(k * 16, 16)
                    m = jnp.logical_and(typbuf[r, sl] == rel, rvalid)
                    tv = tgtbuf[r, sl]
                    plsc.store_compressed(g1d.at[pl.ds(off, 16)],
                                          tv + tv, mask=m)
                    plsc.store_compressed(d1d.at[pl.ds(off, 16)],
                                          srcbuf[r, sl], mask=m)
                    pc = plsc.all_reduce_population_count(m)
                    if pc.ndim:
                        pc = lax.squeeze(lax.slice(pc, (0,), (1,)), (0,))
                    off = off + pc
            return off

        cnt = lax.fori_loop(0, NSUPER, compact, jnp.int32(0))

        padg = (sid * 16 + lanes) * 8
        for k in range(3 * CHUNK // 16):
            g1d[pl.ds(cnt + k * 16, 16)] = padg
            d1d[pl.ds(cnt + k * 16, 16)] = trash
        ntri = (cnt + 383) // jnp.int32(384)
        nchunks = ntri * 3

        plsc.subcore_barrier()

        rows = (rows0, rows1, rows2)
        gsems = (gsem0, gsem1, gsem2)
        ssems = (ssem0, ssem1, ssem2)

        def fire(c, b):
            return pltpu.async_copy(
                hl.at[g1d.at[pl.ds(c * CHUNK, CHUNK)]], rows[b], gsems[b])

        def gdrain(b):
            pltpu.make_async_copy(
                hl.at[g1d.at[pl.ds(0, CHUNK)]], rows[b], gsems[b]).wait()

        def swait(b):
            pltpu.make_async_copy(
                rows[b], acc.at[dwin.at[b]], ssems[b]).wait()

        def chunk_step(i, u, first):
            c = 3 * i + u
            gdrain(u)
            for k in range(CHUNK // 16):
                dwin[u, pl.ds(k * 16, 16)] = d1d[pl.ds(c * CHUNK + k * 16, 16)]
            pltpu.async_copy(rows[u], acc.at[dwin.at[u]], ssems[u], add=True)
            nb = (u + 2) % 3
            if first:
                fire(c + 2, nb)
            else:
                swait(nb)
                fire(jnp.minimum(c + 2, nchunks - 1), nb)

        fire(jnp.int32(0), 0)
        fire(jnp.int32(1), 1)

        @pl.when(ntri > 0)
        def _first_triple():
            chunk_step(jnp.int32(0), 0, True)
            chunk_step(jnp.int32(0), 1, False)
            chunk_step(jnp.int32(0), 2, False)

        def triple(i, carry):
            chunk_step(i, 0, False)
            chunk_step(i, 1, False)
            chunk_step(i, 2, False)
            return carry

        lax.fori_loop(1, ntri, triple, 0)

        gdrain(0)
        gdrain(1)

        @pl.when(ntri > 0)
        def _drain_tail():
            swait(2)

        plsc.subcore_barrier()
        cslice = pl.ds(cid * HID, HID)
        for t in range(3):
            r0 = sid * (3 * OCH) + t * OCH
            pltpu.sync_copy(acc.at[pl.ds(r0, OCH)], obuf)
            pltpu.sync_copy(obuf, out.at[pl.ds(r0, OCH), cslice])

        @pl.when(sid == NS - 1)
        def _copy_tail():
            r0 = NS * 3 * OCH
            pltpu.sync_copy(acc.at[pl.ds(r0, N - r0)],
                            obuf.at[pl.ds(0, N - r0)])
            pltpu.sync_copy(obuf.at[pl.ds(0, N - r0)],
                            out.at[pl.ds(r0, N - r0), cslice])

    return sc_scatter


_sc_scatter = _make_sc_scatter()


def _tc_layer0(x, wcat, bias_full):
    def body(x_ref, w_ref, b_ref, z_ref):
        z_ref[...] = jnp.dot(
            x_ref[...], w_ref[...], preferred_element_type=jnp.float32
        ) + b_ref[...]

    return pl.pallas_call(
        body,
        grid=(N // BR,),
        in_specs=[
            pl.BlockSpec((BR, IN_CH), lambda i: (i, 0)),
            pl.BlockSpec((IN_CH, 2 * HID), lambda i: (0, 0)),
            pl.BlockSpec((1, 2 * HID), lambda i: (0, 0)),
        ],
        out_specs=pl.BlockSpec((BR, 2 * HID), lambda i: (i, 0)),
        out_shape=jax.ShapeDtypeStruct((N, 2 * HID), jnp.float32),
    )(x, wcat, bias_full)


def _tc_mid(parts, zin, wcat, bias_full):
    def body(p_ref, z_ref, w_ref, b_ref, z1_ref):
        h1 = jnp.maximum(p_ref[:, :HID] + p_ref[:, HID:] + z_ref[:, HID:], 0.0)
        z1_ref[...] = jnp.dot(
            h1, w_ref[...], preferred_element_type=jnp.float32
        ) + b_ref[...]

    return pl.pallas_call(
        body,
        grid=(N // BR,),
        in_specs=[
            pl.BlockSpec((BR, 2 * HID), lambda i: (i, 0)),
            pl.BlockSpec((BR, 2 * HID), lambda i: (i, 0)),
            pl.BlockSpec((HID, 2 * HID), lambda i: (0, 0)),
            pl.BlockSpec((1, 2 * HID), lambda i: (0, 0)),
        ],
        out_specs=pl.BlockSpec((BR, 2 * HID), lambda i: (i, 0)),
        out_shape=jax.ShapeDtypeStruct((N, 2 * HID), jnp.float32),
    )(parts, zin, wcat, bias_full)


def _tc_final(parts, zin):
    def body(p_ref, z_ref, o_ref):
        o_ref[...] = jnp.maximum(
            p_ref[:, :HID] + p_ref[:, HID:] + z_ref[:, HID:], 0.0)

    return pl.pallas_call(
        body,
        grid=(N // BR,),
        in_specs=[
            pl.BlockSpec((BR, 2 * HID), lambda i: (i, 0)),
            pl.BlockSpec((BR, 2 * HID), lambda i: (i, 0)),
        ],
        out_specs=pl.BlockSpec((BR, HID), lambda i: (i, 0)),
        out_shape=jax.ShapeDtypeStruct((N, HID), jnp.float32),
    )(parts, zin)


def kernel(x, edge_index, edge_type, w_l0, b_l0, w_00, b_00, w_10, b_10,
           w_l1, b_l1, w_01, b_01, w_11, b_11):
    eidx2 = edge_index.reshape(2 * EROWS, CHUNK)
    typ2 = edge_type.reshape(EROWS, CHUNK)
    zrows = jnp.zeros((OCH, HID), jnp.float32)

    zeros_h = jnp.zeros((HID,), jnp.float32)
    wcat0 = jnp.concatenate([w_l0, w_00 + w_10], axis=1)
    bias0 = jnp.concatenate([zeros_h, b_l0 + b_00 + b_10])[None, :]
    wcat1 = jnp.concatenate([w_l1, w_01 + w_11], axis=1)
    bias1 = jnp.concatenate([zeros_h, b_l1 + b_01 + b_11])[None, :]

    typ2b = typ2 ^ 1

    z0 = _tc_layer0(x, wcat0, bias0)
    parts0 = _sc_scatter(z0.reshape(2 * N, HID), eidx2, typ2, zrows)
    z1 = _tc_mid(parts0, z0, wcat1, bias1)
    parts1 = _sc_scatter(z1.reshape(2 * N, HID), eidx2, typ2b, zrows)
    return _tc_final(parts1, z1)


# speedup vs baseline: 38.5247x; 1.1121x over previous
import functools

import jax
import jax.numpy as jnp
from jax import lax
from jax.experimental import pallas as pl
from jax.experimental.pallas import tpu as pltpu
from jax.experimental.pallas import tpu_sc as plsc

N = 10000
E = 640000
IN_CH = 128
HID = 64

NC = 2
NS = 16
NW = NC * NS
CHUNK = 128
SUPER = 16
NSUPER = 10
EROWS = E // CHUNK
RPW = EROWS // NW
RREM = EROWS - NW * RPW

ACC_ROWS = 10144
TRASH0 = 10016
OCH = 156

EDGES_PER_W = (RPW + 1) * CHUNK
CBUF = EDGES_PER_W + 4 * CHUNK

BR = 2000


def _make_sc_scatter():
    rel = 0
    mesh = plsc.VectorSubcoreMesh(
        core_axis_name="c", subcore_axis_name="s", num_cores=NC, num_subcores=NS
    )

    @functools.partial(
        pl.kernel,
        out_type=jax.ShapeDtypeStruct((N, NC * HID), jnp.float32),
        mesh=mesh,
        compiler_params=pltpu.CompilerParams(
            use_tc_tiling_on_sc=False, needs_layout_passes=False),
        scratch_types=[
            pltpu.VMEM((2, SUPER, CHUNK), jnp.int32),
            pltpu.VMEM((2, SUPER, CHUNK), jnp.int32),
            pltpu.VMEM((2, SUPER, CHUNK), jnp.int32),
            pltpu.VMEM((CBUF,), jnp.int32),
            pltpu.VMEM((CBUF,), jnp.int32),
            pltpu.VMEM((3, CHUNK), jnp.int32),
            pltpu.VMEM((CHUNK, HID), jnp.float32),
            pltpu.VMEM((CHUNK, HID), jnp.float32),
            pltpu.VMEM((CHUNK, HID), jnp.float32),
            pltpu.VMEM((OCH, HID), jnp.float32),
            pltpu.VMEM_SHARED((ACC_ROWS, HID), jnp.float32),
            pltpu.SemaphoreType.DMA,
            pltpu.SemaphoreType.DMA,
            pltpu.SemaphoreType.DMA,
            pltpu.SemaphoreType.DMA,
            pltpu.SemaphoreType.DMA,
            pltpu.SemaphoreType.DMA,
            pltpu.SemaphoreType.DMA,
            pltpu.SemaphoreType.DMA,
        ],
    )
    def sc_scatter(hl, eidx2, typ2, zrows, out,
                   tgtbuf, srcbuf, typbuf, g1d, d1d, dwin, rows0, rows1, rows2,
                   obuf, acc, gsem0, gsem1, gsem2, ssem0, ssem1, ssem2,
                   lsem0, lsem1):
        cid = lax.axis_index("c")
        sid = lax.axis_index("s")
        wid = sid * NC + cid

        zpt = ACC_ROWS // NS
        pltpu.sync_copy(zrows, obuf)
        for t in range(4):
            pltpu.sync_copy(obuf, acc.at[pl.ds(sid * zpt + t * OCH, OCH)])
        pltpu.sync_copy(obuf.at[pl.ds(0, zpt - 4 * OCH)],
                        acc.at[pl.ds(sid * zpt + 4 * OCH, zpt - 4 * OCH)])

        lanes = lax.iota(jnp.int32, 16)
        trash = TRASH0 + sid * 8 + (lanes & 7)

        c0 = wid * RPW + jnp.minimum(wid, RREM)
        nw = RPW + jnp.where(wid < RREM, 1, 0)
        lsems = (lsem0, lsem1)

        def lbase(s):
            return jnp.minimum(c0 + SUPER * s, c0 + nw - SUPER)

        def fire_loads(s, b):
            base = lbase(s)
            pltpu.async_copy(eidx2.at[pl.ds(EROWS + base, SUPER)],
                             tgtbuf.at[b], lsems[b])
            pltpu.async_copy(eidx2.at[pl.ds(base, SUPER)],
                             srcbuf.at[b], lsems[b])
            pltpu.async_copy(typ2.at[pl.ds(base, SUPER)],
                             typbuf.at[b], lsems[b])

        def wait_loads(b):
            for _ in range(3):
                pltpu.make_async_copy(typ2.at[pl.ds(0, SUPER)],
                                      typbuf.at[b], lsems[b]).wait()

        def compact_one(s, b, off):
            base = lbase(s)
            lim = c0 + SUPER * s
            for r in range(SUPER):
                rvalid = base + r >= lim
                for k in range(CHUNK // 16):
                    sl = pl.ds(k * 16, 16)
                    m = jnp.logical_and(typbuf[b, r, sl] == rel, rvalid)
                    tv = tgtbuf[b, r, sl]
                    plsc.store_compressed(g1d.at[pl.ds(off, 16)],
                                          tv + tv, mask=m)
                    plsc.store_compressed(d1d.at[pl.ds(off, 16)],
                                          srcbuf[b, r, sl], mask=m)
                    pc = plsc.all_reduce_population_count(m)
                    if pc.ndim:
                        pc = lax.squeeze(lax.slice(pc, (0,), (1,)), (0,))
                    off = off + pc
            return off

        fire_loads(jnp.int32(0), 0)

        def cpair(i, off):
            fire_loads(2 * i + 1, 1)
            wait_loads(0)
            off = compact_one(2 * i, 0, off)
            fire_loads(jnp.minimum(2 * i + 2, NSUPER - 1), 0)
            wait_loads(1)
            off = compact_one(2 * i + 1, 1, off)
            return off

        cnt = lax.fori_loop(0, NSUPER // 2, cpair, jnp.int32(0))
        wait_loads(0)

        padg = (sid * 16 + lanes) * 8
        for k in range(3 * CHUNK // 16):
            g1d[pl.ds(cnt + k * 16, 16)] = padg
            d1d[pl.ds(cnt + k * 16, 16)] = trash
        ntri = (cnt + 383) // jnp.int32(384)
        nchunks = ntri * 3

        plsc.subcore_barrier()

        rows = (rows0, rows1, rows2)
        gsems = (gsem0, gsem1, gsem2)
        ssems = (ssem0, ssem1, ssem2)

        def fire(c, b):
            return pltpu.async_copy(
                hl.at[g1d.at[pl.ds(c * CHUNK, CHUNK)]], rows[b], gsems[b])

        def gdrain(b):
            pltpu.make_async_copy(
                hl.at[g1d.at[pl.ds(0, CHUNK)]], rows[b], gsems[b]).wait()

        def swait(b):
            pltpu.make_async_copy(
                rows[b], acc.at[dwin.at[b]], ssems[b]).wait()

        def chunk_step(i, u, first):
            c = 3 * i + u
            for k in range(CHUNK // 16):
                dwin[u, pl.ds(k * 16, 16)] = d1d[pl.ds(c * CHUNK + k * 16, 16)]
            gdrain(u)
            pltpu.async_copy(rows[u], acc.at[dwin.at[u]], ssems[u], add=True)
            nb = (u + 2) % 3
            if first:
                fire(c + 2, nb)
            else:
                swait(nb)
                fire(jnp.minimum(c + 2, nchunks - 1), nb)

        fire(jnp.int32(0), 0)
        fire(jnp.int32(1), 1)

        @pl.when(ntri > 0)
        def _first_triple():
            chunk_step(jnp.int32(0), 0, True)
            chunk_step(jnp.int32(0), 1, False)
            chunk_step(jnp.int32(0), 2, False)

        def triple(i, carry):
            chunk_step(i, 0, False)
            chunk_step(i, 1, False)
            chunk_step(i, 2, False)
            return carry

        lax.fori_loop(1, ntri, triple, 0)

        gdrain(0)
        gdrain(1)

        @pl.when(ntri > 0)
        def _drain_tail():
            swait(2)

        plsc.subcore_barrier()
        cslice = pl.ds(cid * HID, HID)
        for t in range(4):
            r0 = sid * (4 * OCH) + t * OCH
            pltpu.sync_copy(acc.at[pl.ds(r0, OCH)], obuf)
            pltpu.sync_copy(obuf, out.at[pl.ds(r0, OCH), cslice])

        @pl.when(sid == NS - 1)
        def _copy_tail():
            r0 = NS * 4 * OCH
            pltpu.sync_copy(acc.at[pl.ds(r0, N - r0)],
                            obuf.at[pl.ds(0, N - r0)])
            pltpu.sync_copy(obuf.at[pl.ds(0, N - r0)],
                            out.at[pl.ds(r0, N - r0), cslice])

    return sc_scatter


_sc_scatter = _make_sc_scatter()


def _tc_layer0(x, wcat, bias_full):
    def body(x_ref, w_ref, b_ref, z_ref):
        z_ref[...] = jnp.dot(
            x_ref[...], w_ref[...], preferred_element_type=jnp.float32
        ) + b_ref[...]

    return pl.pallas_call(
        body,
        grid=(N // BR,),
        in_specs=[
            pl.BlockSpec((BR, IN_CH), lambda i: (i, 0)),
            pl.BlockSpec((IN_CH, 2 * HID), lambda i: (0, 0)),
            pl.BlockSpec((1, 2 * HID), lambda i: (0, 0)),
        ],
        out_specs=pl.BlockSpec((BR, 2 * HID), lambda i: (i, 0)),
        out_shape=jax.ShapeDtypeStruct((N, 2 * HID), jnp.float32),
    )(x, wcat, bias_full)


def _tc_mid(parts, zin, wcat, bias_full):
    def body(p_ref, z_ref, w_ref, b_ref, z1_ref):
        h1 = jnp.maximum(p_ref[:, :HID] + p_ref[:, HID:] + z_ref[:, HID:], 0.0)
        z1_ref[...] = jnp.dot(
            h1, w_ref[...], preferred_element_type=jnp.float32
        ) + b_ref[...]

    return pl.pallas_call(
        body,
        grid=(N // BR,),
        in_specs=[
            pl.BlockSpec((BR, 2 * HID), lambda i: (i, 0)),
            pl.BlockSpec((BR, 2 * HID), lambda i: (i, 0)),
            pl.BlockSpec((HID, 2 * HID), lambda i: (0, 0)),
            pl.BlockSpec((1, 2 * HID), lambda i: (0, 0)),
        ],
        out_specs=pl.BlockSpec((BR, 2 * HID), lambda i: (i, 0)),
        out_shape=jax.ShapeDtypeStruct((N, 2 * HID), jnp.float32),
    )(parts, zin, wcat, bias_full)


def _tc_final(parts, zin):
    def body(p_ref, z_ref, o_ref):
        o_ref[...] = jnp.maximum(
            p_ref[:, :HID] + p_ref[:, HID:] + z_ref[:, HID:], 0.0)

    return pl.pallas_call(
        body,
        grid=(N // BR,),
        in_specs=[
            pl.BlockSpec((BR, 2 * HID), lambda i: (i, 0)),
            pl.BlockSpec((BR, 2 * HID), lambda i: (i, 0)),
        ],
        out_specs=pl.BlockSpec((BR, HID), lambda i: (i, 0)),
        out_shape=jax.ShapeDtypeStruct((N, HID), jnp.float32),
    )(parts, zin)


def kernel(x, edge_index, edge_type, w_l0, b_l0, w_00, b_00, w_10, b_10,
           w_l1, b_l1, w_01, b_01, w_11, b_11):
    eidx2 = edge_index.reshape(2 * EROWS, CHUNK)
    typ2 = edge_type.reshape(EROWS, CHUNK)
    zrows = jnp.zeros((OCH, HID), jnp.float32)

    zeros_h = jnp.zeros((HID,), jnp.float32)
    wcat0 = jnp.concatenate([w_l0, w_00 + w_10], axis=1)
    bias0 = jnp.concatenate([zeros_h, b_l0 + b_00 + b_10])[None, :]
    wcat1 = jnp.concatenate([w_l1, w_01 + w_11], axis=1)
    bias1 = jnp.concatenate([zeros_h, b_l1 + b_01 + b_11])[None, :]

    typ2b = typ2 ^ 1

    z0 = _tc_layer0(x, wcat0, bias0)
    parts0 = _sc_scatter(z0.reshape(2 * N, HID), eidx2, typ2, zrows)
    z1 = _tc_mid(parts0, z0, wcat1, bias1)
    parts1 = _sc_scatter(z1.reshape(2 * N, HID), eidx2, typ2b, zrows)
    return _tc_final(parts1, z1)
